# Initial kernel scaffold; baseline (speedup 1.0000x reference)
import jax
import jax.numpy as jnp
from jax.experimental import pallas as pl


def kernel(x, embedding, new_embedding):
    raise NotImplementedError("write your pallas kernel here")


# speedup vs baseline: 1.0859x; 1.0859x over previous
import functools

import jax
import jax.numpy as jnp
from jax import lax
from jax.experimental import pallas as pl
from jax.experimental.pallas import tpu as pltpu
from jax.experimental.pallas import tpu_sc as plsc


def _make_gather(n_main, n_new, d, batch):
    info = plsc.get_sparse_core_info()
    nc, ns, nl = info.num_cores, info.num_subcores, info.num_lanes
    nw = nc * ns
    assert batch % nw == 0
    b_per_w = batch // nw
    CHUNK = 2048 if b_per_w % 2048 == 0 else b_per_w
    assert b_per_w % CHUNK == 0
    GROUP = 128
    assert CHUNK % GROUP == 0
    n_feat_chunks = d // nl

    mesh = plsc.VectorSubcoreMesh(core_axis_name="c", subcore_axis_name="s")

    @functools.partial(
        pl.kernel,
        mesh=mesh,
        out_type=jax.ShapeDtypeStruct((batch, d), jnp.float32),
        compiler_params=pltpu.CompilerParams(
            use_tc_tiling_on_sc=False, needs_layout_passes=False
        ),
        scratch_types=[
            pltpu.VMEM((CHUNK,), jnp.int32),
            pltpu.VMEM((CHUNK,), jnp.int32),
            pltpu.VMEM((GROUP, d), jnp.float32),
            pltpu.VMEM((n_new * d,), jnp.float32),
            pltpu.VMEM((nl,), jnp.int32),
            pltpu.SemaphoreType.DMA,
        ],
    )
    def gather_kernel(emb_hbm, new_hbm, idx_hbm, out_hbm,
                      idx_v, midx_v, rows_v, new_v, maxacc_v, sem):
        wid = lax.axis_index("s") * nc + lax.axis_index("c")
        base = wid * b_per_w
        pltpu.sync_copy(new_hbm, new_v)

        def chunk_body(c, _):
            cbase = base + c * CHUNK
            pltpu.sync_copy(idx_hbm.at[pl.ds(cbase, CHUNK)], idx_v)
            maxacc_v[...] = jnp.zeros((nl,), jnp.int32)

            def clamp_body(s, _):
                v = idx_v[pl.ds(s * nl, nl)]
                midx_v[pl.ds(s * nl, nl)] = jnp.minimum(v, n_main - 1)
                maxacc_v[...] = jnp.maximum(maxacc_v[...], v)
                return 0

            lax.fori_loop(0, CHUNK // nl, clamp_body, 0)
            chunk_has_over = jnp.max(maxacc_v[...]) >= n_main

            def group_body(g, _):
                goff = g * GROUP
                pltpu.async_copy(
                    emb_hbm.at[midx_v.at[pl.ds(goff, GROUP)]], rows_v, sem
                ).wait()

                @pl.when(chunk_has_over)
                def _():
                    def fix_body(s, _):
                        off = goff + s * nl
                        smax = jnp.max(idx_v[pl.ds(off, nl)])

                        @pl.when(smax >= n_main)
                        def _():
                            v = idx_v[pl.ds(off, nl)]
                            m = v >= n_main
                            nidx = jnp.clip(v - n_main, 0, n_new - 1)
                            lane = jnp.arange(nl, dtype=jnp.int32)
                            rows_ids = s * nl + lane

                            def feat_body(f, _):
                                colf = jnp.full((nl,), 0, jnp.int32) + f
                                vals = plsc.load_gather(
                                    new_v, [nidx * d + colf]
                                )
                                plsc.store_scatter(
                                    rows_v, [rows_ids, colf], vals, mask=m
                                )
                                return 0

                            lax.fori_loop(0, d, feat_body, 0)

                        return 0

                    lax.fori_loop(0, GROUP // nl, fix_body, 0)

                pltpu.sync_copy(rows_v, out_hbm.at[pl.ds(cbase + goff, GROUP)])
                return 0

            lax.fori_loop(0, CHUNK // GROUP, group_body, 0)
            return 0

        lax.fori_loop(0, b_per_w // CHUNK, chunk_body, 0)

    return gather_kernel


def kernel(x, embedding, new_embedding):
    n_main, d = embedding.shape
    n_new = new_embedding.shape[0]
    b, h = x.shape
    batch = b * h
    idx = x.reshape(-1).astype(jnp.int32)
    gather = _make_gather(n_main, n_new, d, batch)
    out = gather(embedding, new_embedding.reshape(-1), idx)
    return out.reshape(b, h, d)


# speedup vs baseline: 1.1801x; 1.0868x over previous
import functools

import jax
import jax.numpy as jnp
from jax import lax
from jax.experimental import pallas as pl
from jax.experimental.pallas import tpu as pltpu
from jax.experimental.pallas import tpu_sc as plsc


def _make_gather(n_main, n_new, d, batch):
    info = plsc.get_sparse_core_info()
    nc, ns, nl = info.num_cores, info.num_subcores, info.num_lanes
    nw = nc * ns
    assert batch % nw == 0
    b_per_w = batch // nw
    GROUP = 128
    assert b_per_w % GROUP == 0
    n_groups = b_per_w // GROUP
    NBUF = 8
    assert n_groups % NBUF == 0
    sub_per_group = GROUP // nl

    mesh = plsc.VectorSubcoreMesh(core_axis_name="c", subcore_axis_name="s")

    @functools.partial(
        pl.kernel,
        mesh=mesh,
        out_type=jax.ShapeDtypeStruct((batch, d), jnp.float32),
        compiler_params=pltpu.CompilerParams(
            use_tc_tiling_on_sc=False, needs_layout_passes=False
        ),
        scratch_types=[
            pltpu.VMEM((b_per_w,), jnp.int32),
            pltpu.VMEM((b_per_w,), jnp.int32),
            pltpu.VMEM((n_groups * nl,), jnp.int32),
            pltpu.VMEM((NBUF, GROUP, d), jnp.float32),
            pltpu.VMEM((n_new * d,), jnp.float32),
        ]
        + [pltpu.SemaphoreType.DMA] * (2 * NBUF),
    )
    def gather_kernel(emb_hbm, new_hbm, idx_hbm, out_hbm,
                      idx_v, midx_v, gmax_v, rows_v, new_v, *sems):
        gsems, ssems = sems[:NBUF], sems[NBUF:]
        wid = lax.axis_index("s") * nc + lax.axis_index("c")
        base = wid * b_per_w
        pltpu.sync_copy(new_hbm, new_v)
        pltpu.sync_copy(idx_hbm.at[pl.ds(base, b_per_w)], idx_v)

        def block_body(gb, _):
            copies = []
            for b in range(NBUF):
                g = gb * NBUF + b
                goff = g * GROUP

                @pl.when(gb > 0)
                def _():
                    pltpu.make_async_copy(
                        rows_v.at[b], out_hbm.at[pl.ds(0, GROUP)], ssems[b]
                    ).wait()

                gmax = None
                for s in range(sub_per_group):
                    v = idx_v[pl.ds(goff + s * nl, nl)]
                    midx_v[pl.ds(goff + s * nl, nl)] = jnp.minimum(
                        v, n_main - 1
                    )
                    gmax = v if gmax is None else jnp.maximum(gmax, v)
                gmax_v[pl.ds(g * nl, nl)] = gmax
                copies.append(
                    pltpu.async_copy(
                        emb_hbm.at[midx_v.at[pl.ds(goff, GROUP)]],
                        rows_v.at[b],
                        gsems[b],
                    )
                )

            for b in range(NBUF):
                g = gb * NBUF + b
                goff = g * GROUP
                copies[b].wait()
                gmax_s = jnp.max(gmax_v[pl.ds(g * nl, nl)])

                @pl.when(gmax_s >= n_main)
                def _():
                    def fix_body(s, _):
                        off = goff + s * nl
                        smax = jnp.max(idx_v[pl.ds(off, nl)])

                        @pl.when(smax >= n_main)
                        def _():
                            v = idx_v[pl.ds(off, nl)]
                            m = v >= n_main
                            nidx = jnp.clip(v - n_main, 0, n_new - 1)
                            lane = jnp.arange(nl, dtype=jnp.int32)
                            bvec = jnp.full((nl,), b, jnp.int32)
                            rows_ids = s * nl + lane

                            def feat_body(f, _):
                                colf = jnp.full((nl,), 0, jnp.int32) + f
                                vals = plsc.load_gather(
                                    new_v, [nidx * d + colf]
                                )
                                plsc.store_scatter(
                                    rows_v,
                                    [bvec, rows_ids, colf],
                                    vals,
                                    mask=m,
                                )
                                return 0

                            lax.fori_loop(0, d, feat_body, 0)

                        return 0

                    lax.fori_loop(0, sub_per_group, fix_body, 0)

                pltpu.async_copy(
                    rows_v.at[b],
                    out_hbm.at[pl.ds(base + goff, GROUP)],
                    ssems[b],
                )
            return 0

        lax.fori_loop(0, n_groups // NBUF, block_body, 0)
        for b in range(NBUF):
            pltpu.make_async_copy(
                rows_v.at[b], out_hbm.at[pl.ds(0, GROUP)], ssems[b]
            ).wait()

    return gather_kernel


def kernel(x, embedding, new_embedding):
    n_main, d = embedding.shape
    n_new = new_embedding.shape[0]
    b, h = x.shape
    batch = b * h
    idx = x.reshape(-1).astype(jnp.int32)
    gather = _make_gather(n_main, n_new, d, batch)
    out = gather(embedding, new_embedding.reshape(-1), idx)
    return out.reshape(b, h, d)
